# baseline (speedup 1.0000x reference)
import functools

import jax
import jax.numpy as jnp
from jax import lax
from jax.experimental import pallas as pl
from jax.experimental.pallas import tpu as pltpu
from jax.experimental.pallas import tpu_sc as plsc

L = 16
KC_BLK = 2048


def _kc_of_body(At_ref, out_ref):
    a = At_ref[...]
    ramp = lax.broadcasted_iota(jnp.int32, a.shape, 0).astype(jnp.float32)
    out_ref[...] = jnp.sum(a * ramp, axis=0).astype(jnp.int32)


def _compute_kc_of(At):
    K, V = At.shape
    grid = (pl.cdiv(V, KC_BLK),)
    return pl.pallas_call(
        _kc_of_body,
        grid=grid,
        in_specs=[pl.BlockSpec((K, KC_BLK), lambda i: (0, i))],
        out_specs=pl.BlockSpec((KC_BLK,), lambda i: (i,)),
        out_shape=jax.ShapeDtypeStruct((V,), jnp.int32),
    )(At)


def _sc_gather_ids(table, idx):
    N = idx.shape[0]
    V = table.shape[0]
    info = plsc.get_sparse_core_info()
    nw = info.num_cores * info.num_subcores
    n_per_w = N // nw
    mesh = plsc.VectorSubcoreMesh(core_axis_name="c", subcore_axis_name="s")

    @functools.partial(
        pl.kernel,
        mesh=mesh,
        out_type=jax.ShapeDtypeStruct((N,), jnp.int32),
        compiler_params=pltpu.CompilerParams(needs_layout_passes=False),
        scratch_types=[
            pltpu.VMEM((V,), jnp.int32),
            pltpu.VMEM((n_per_w,), jnp.int32),
            pltpu.VMEM((n_per_w,), jnp.int32),
        ],
    )
    def k(table_hbm, idx_hbm, out_hbm, table_v, idx_v, out_v):
        wid = lax.axis_index("s") * info.num_cores + lax.axis_index("c")
        base = wid * n_per_w
        pltpu.sync_copy(table_hbm, table_v)
        pltpu.sync_copy(idx_hbm.at[pl.ds(base, n_per_w)], idx_v)
        for i in range(n_per_w // L):
            ivec = idx_v[pl.ds(i * L, L)]
            out_v[pl.ds(i * L, L)] = plsc.load_gather(table_v, [ivec])
        pltpu.sync_copy(out_v, out_hbm.at[pl.ds(base, n_per_w)])

    return k(table, idx)


def _bkt_body(ids_ref, pcor_ref, Wt_ref, At_ref, probs_ref, state_ref):
    T = pcor_ref.shape[0]
    blk = pcor_ref.shape[1]
    K = Wt_ref.shape[1]
    i = pl.program_id(0)
    B2 = ids_ref.shape[0] // T
    sw = jax.nn.sigmoid(Wt_ref[...].T)
    ramp = lax.broadcasted_iota(jnp.int32, (K, blk), 0)

    def row(base):
        return ids_ref[pl.ds(pl.multiple_of(base, blk), blk)].reshape(1, blk)

    def wsel(oh, c):
        return jnp.sum(oh * sw[:, c:c + 1], axis=0, keepdims=True)

    state = jnp.broadcast_to(sw[:, 4:5], (K, blk))
    for t in range(T):
        oc = (ramp == row(t * B2 + B2 // 2 + i * blk)).astype(jnp.float32)
        c2 = wsel(oc, 2)
        c3 = wsel(oc, 3)
        if t > 0:
            opb = ramp == row(t * B2 + i * blk)
            op = opb.astype(jnp.float32)
            p0 = wsel(op, 0)
            p1 = wsel(op, 1)
            p2 = wsel(op, 2)
            p3 = wsel(op, 3)
            pcor = pcor_ref[t:t + 1, :]
            ss = jnp.sum(state * op, axis=0, keepdims=True)
            po0 = jnp.where(pcor > 0.5, p2, 1.0 - p2)
            po1 = jnp.where(pcor > 0.5, p3, 1.0 - p3)
            filt = po1 * ss / (po0 * (1.0 - ss) + po1 * ss)
            pred = p0 * (1.0 - filt) + (1.0 - p1) * filt
            state = jnp.where(opb, pred, state)
        cs = jnp.sum(state * oc, axis=0, keepdims=True)
        probs_ref[t:t + 1, :] = c2 * (1.0 - cs) + c3 * cs
    state_ref[...] = jax.lax.dot_general(
        At_ref[...], state, (((0,), (0,)), ((), ())),
        preferred_element_type=jnp.float32)


def kernel(prev_kc, curr_kc, prev_corr, A, W):
    B, T = prev_kc.shape
    V, K = A.shape
    At = A.T

    kc_of = _compute_kc_of(At)
    idx = jnp.concatenate(
        [prev_kc, curr_kc], axis=0).T.reshape(-1).astype(jnp.int32)
    ids = _sc_gather_ids(kc_of, idx)

    blk = 512
    probsT, stateT = pl.pallas_call(
        _bkt_body,
        grid=(B // blk,),
        in_specs=[
            pl.BlockSpec((2 * B * T,), lambda i: (0,)),
            pl.BlockSpec((T, blk), lambda i: (0, i)),
            pl.BlockSpec((5, K), lambda i: (0, 0)),
            pl.BlockSpec((K, V), lambda i: (0, 0)),
        ],
        out_specs=[
            pl.BlockSpec((T, blk), lambda i: (0, i)),
            pl.BlockSpec((V, blk), lambda i: (0, i)),
        ],
        out_shape=[
            jax.ShapeDtypeStruct((T, B), jnp.float32),
            jax.ShapeDtypeStruct((V, B), jnp.float32),
        ],
        compiler_params=pltpu.CompilerParams(
            fuse_transposed_lhs_in_matmul=True),
    )(ids, prev_corr.T, W.T, At)
    return probsT.T, stateT.T


# speedup vs baseline: 1.0562x; 1.0562x over previous
import functools

import jax
import jax.numpy as jnp
from jax import lax
from jax.experimental import pallas as pl
from jax.experimental.pallas import tpu as pltpu
from jax.experimental.pallas import tpu_sc as plsc

L = 16
KC_BLK = 2048


def _kc_of_body(At_ref, out_ref):
    a = At_ref[...]
    ramp = lax.broadcasted_iota(jnp.int32, a.shape, 0).astype(jnp.float32)
    out_ref[...] = jnp.sum(a * ramp, axis=0).astype(jnp.int32)


def _compute_kc_of(At):
    K, V = At.shape
    grid = (pl.cdiv(V, KC_BLK),)
    return pl.pallas_call(
        _kc_of_body,
        grid=grid,
        in_specs=[pl.BlockSpec((K, KC_BLK), lambda i: (0, i))],
        out_specs=pl.BlockSpec((KC_BLK,), lambda i: (i,)),
        out_shape=jax.ShapeDtypeStruct((V,), jnp.int32),
    )(At)


def _sc_gather_ids(table, idx):
    N = idx.shape[0]
    V = table.shape[0]
    info = plsc.get_sparse_core_info()
    nw = info.num_cores * info.num_subcores
    n_per_w = N // nw
    mesh = plsc.VectorSubcoreMesh(core_axis_name="c", subcore_axis_name="s")

    @functools.partial(
        pl.kernel,
        mesh=mesh,
        out_type=jax.ShapeDtypeStruct((N,), jnp.int32),
        compiler_params=pltpu.CompilerParams(needs_layout_passes=False),
        scratch_types=[
            pltpu.VMEM((V,), jnp.int32),
            pltpu.VMEM((n_per_w,), jnp.int32),
            pltpu.VMEM((n_per_w,), jnp.int32),
        ],
    )
    def k(table_hbm, idx_hbm, out_hbm, table_v, idx_v, out_v):
        wid = lax.axis_index("s") * info.num_cores + lax.axis_index("c")
        base = wid * n_per_w
        pltpu.sync_copy(table_hbm, table_v)
        pltpu.sync_copy(idx_hbm.at[pl.ds(base, n_per_w)], idx_v)
        for i in range(n_per_w // L):
            ivec = idx_v[pl.ds(i * L, L)]
            out_v[pl.ds(i * L, L)] = plsc.load_gather(table_v, [ivec])
        pltpu.sync_copy(out_v, out_hbm.at[pl.ds(base, n_per_w)])

    return k(table, idx)


def _bkt_body(ids_ref, pcor_ref, Wt_ref, At_ref, probs_ref, state_ref):
    T = pcor_ref.shape[0]
    blk = pcor_ref.shape[1]
    K = Wt_ref.shape[1]
    i = pl.program_id(0)
    B2 = ids_ref.shape[0] // T
    sw = jax.nn.sigmoid(Wt_ref[...].T)
    ramp = lax.broadcasted_iota(jnp.int32, (K, blk), 0)

    def row(base):
        return ids_ref[pl.ds(pl.multiple_of(base, blk), blk)].reshape(1, blk)

    def wsel(oh, c):
        return jnp.sum(oh * sw[:, c:c + 1], axis=0, keepdims=True)

    state = jnp.broadcast_to(sw[:, 4:5], (K, blk))
    for t in range(T):
        oc = (ramp == row(t * B2 + B2 // 2 + i * blk)).astype(jnp.float32)
        c2 = wsel(oc, 2)
        c3 = wsel(oc, 3)
        if t > 0:
            opb = ramp == row(t * B2 + i * blk)
            op = opb.astype(jnp.float32)
            p0 = wsel(op, 0)
            p1 = wsel(op, 1)
            p2 = wsel(op, 2)
            p3 = wsel(op, 3)
            pcor = pcor_ref[t:t + 1, :]
            ss = jnp.sum(state * op, axis=0, keepdims=True)
            po0 = jnp.where(pcor > 0.5, p2, 1.0 - p2)
            po1 = jnp.where(pcor > 0.5, p3, 1.0 - p3)
            filt = po1 * ss / (po0 * (1.0 - ss) + po1 * ss)
            pred = p0 * (1.0 - filt) + (1.0 - p1) * filt
            state = jnp.where(opb, pred, state)
        cs = jnp.sum(state * oc, axis=0, keepdims=True)
        probs_ref[t:t + 1, :] = c2 * (1.0 - cs) + c3 * cs
    state_ref[...] = jax.lax.dot_general(
        At_ref[...], state, (((0,), (0,)), ((), ())),
        preferred_element_type=jnp.float32)


def kernel(prev_kc, curr_kc, prev_corr, A, W):
    B, T = prev_kc.shape
    V, K = A.shape
    At = A.T

    kc_of = _compute_kc_of(At)
    idx = jnp.concatenate(
        [prev_kc, curr_kc], axis=0).T.reshape(-1).astype(jnp.int32)
    ids = _sc_gather_ids(kc_of, idx)

    blk = 256
    probsT, stateT = pl.pallas_call(
        _bkt_body,
        grid=(B // blk,),
        in_specs=[
            pl.BlockSpec((2 * B * T,), lambda i: (0,)),
            pl.BlockSpec((T, blk), lambda i: (0, i)),
            pl.BlockSpec((5, K), lambda i: (0, 0)),
            pl.BlockSpec((K, V), lambda i: (0, 0)),
        ],
        out_specs=[
            pl.BlockSpec((T, blk), lambda i: (0, i)),
            pl.BlockSpec((V, blk), lambda i: (0, i)),
        ],
        out_shape=[
            jax.ShapeDtypeStruct((T, B), jnp.float32),
            jax.ShapeDtypeStruct((V, B), jnp.float32),
        ],
        compiler_params=pltpu.CompilerParams(
            fuse_transposed_lhs_in_matmul=True),
    )(ids, prev_corr.T, W.T, At)
    return probsT.T, stateT.T
